# Initial kernel scaffold; baseline (speedup 1.0000x reference)
import jax
import jax.numpy as jnp
from jax.experimental import pallas as pl


def kernel(hidden_states, router_weight, w_gate_up, w_down, shared_gate_up, shared_down):
    raise NotImplementedError("write your pallas kernel here")


# speedup vs baseline: 1.4234x; 1.4234x over previous
import functools

import jax
import jax.numpy as jnp
from jax.experimental import pallas as pl
from jax.experimental.pallas import tpu as pltpu

T = 2048
H = 1024
E = 8
K = 2
I = 512
BM = 256


def _moe_body(x_ref, rwt_ref, wgu_ref, wd_ref, sgu_ref, sd_ref, o_ref):
    x = x_ref[...]

    logits = jax.lax.dot_general(
        x, rwt_ref[...], (((1,), (0,)), ((), ())),
        precision=jax.lax.Precision.DEFAULT,
        preferred_element_type=jnp.float32,
    )
    m = jnp.max(logits, axis=-1, keepdims=True)
    ex = jnp.exp(logits - m)
    probs = ex / jnp.sum(ex, axis=-1, keepdims=True)

    lane = jax.lax.broadcasted_iota(jnp.int32, probs.shape, 1)
    p1 = jnp.max(probs, axis=-1, keepdims=True)
    i1 = jnp.min(jnp.where(probs == p1, lane, E), axis=-1, keepdims=True)
    mask1 = lane == i1
    probs_rest = jnp.where(mask1, -jnp.inf, probs)
    p2 = jnp.max(probs_rest, axis=-1, keepdims=True)
    i2 = jnp.min(jnp.where(probs_rest == p2, lane, E), axis=-1, keepdims=True)
    mask2 = lane == i2
    denom = p1 + p2
    combine = (jnp.where(mask1, p1, 0.0) + jnp.where(mask2, p2, 0.0)) / denom

    xb = x.astype(jnp.bfloat16)

    def mlp(w_gu, w_d):
        gu = jax.lax.dot_general(
            xb, w_gu, (((1,), (0,)), ((), ())),
            preferred_element_type=jnp.float32)
        g = gu[:, :I]
        u = gu[:, I:]
        act = (g / (1.0 + jnp.exp(-g))) * u
        return jax.lax.dot_general(
            act.astype(jnp.bfloat16), w_d, (((1,), (0,)), ((), ())),
            preferred_element_type=jnp.float32)

    acc = mlp(sgu_ref[...], sd_ref[...])
    for e in range(E):
        ye = mlp(wgu_ref[e], wd_ref[e])
        acc = acc + combine[:, e:e + 1] * ye
    o_ref[...] = acc


@jax.jit
def kernel(hidden_states, router_weight, w_gate_up, w_down, shared_gate_up,
           shared_down):
    rwt = router_weight.T.astype(jnp.float32)
    wgu = w_gate_up.astype(jnp.bfloat16)
    wd = w_down.astype(jnp.bfloat16)
    sgu = shared_gate_up.astype(jnp.bfloat16)
    sd = shared_down.astype(jnp.bfloat16)

    grid = (T // BM,)
    out = pl.pallas_call(
        _moe_body,
        grid=grid,
        in_specs=[
            pl.BlockSpec((BM, H), lambda i: (i, 0)),
            pl.BlockSpec((H, E), lambda i: (0, 0)),
            pl.BlockSpec((E, H, 2 * I), lambda i: (0, 0, 0)),
            pl.BlockSpec((E, I, H), lambda i: (0, 0, 0)),
            pl.BlockSpec((H, 2 * I), lambda i: (0, 0)),
            pl.BlockSpec((I, H), lambda i: (0, 0)),
        ],
        out_specs=pl.BlockSpec((BM, H), lambda i: (i, 0)),
        out_shape=jax.ShapeDtypeStruct((T, H), jnp.float32),
    )(hidden_states, rwt, wgu, wd, sgu, sd)
    return out
